# Initial kernel scaffold; baseline (speedup 1.0000x reference)
import jax
import jax.numpy as jnp
from jax.experimental import pallas as pl


def kernel(inputs, table, W1, b1):
    raise NotImplementedError("write your pallas kernel here")


# speedup vs baseline: 1.4872x; 1.4872x over previous
import jax
import jax.numpy as jnp
from jax.experimental import pallas as pl
from jax.experimental.pallas import tpu as pltpu
from jax.experimental.pallas import tpu_sc as plsc

_EMB = 128
_LIN = 256
_GW = 128
_BM = 2048


def _matmul_relu_block(x_ref, w_ref, b_ref, o_ref):
    acc = jnp.dot(x_ref[...], w_ref[...], preferred_element_type=jnp.float32)
    o_ref[...] = jnp.maximum(acc + b_ref[...], 0.0)


def kernel(inputs, table, W1, b1):
    B, L = inputs.shape
    n = B * L
    idx = inputs.reshape(1, n).astype(jnp.int32)

    mesh = plsc.VectorSubcoreMesh(core_axis_name="core", subcore_axis_name="subcore")

    @pl.kernel(out_type=jax.ShapeDtypeStruct((n, _EMB), table.dtype), mesh=mesh)
    def gather_rows(table_hbm, idx_hbm, out_hbm):
        def body(idx_vmem, out_vmem):
            pltpu.sync_copy(table_hbm.at[idx_vmem.at[0]], out_vmem)

        pltpu.emit_pipeline(
            body,
            grid=(n // _GW,),
            in_specs=[pl.BlockSpec((1, _GW), index_map=lambda i: (0, i))],
            out_specs=[pl.BlockSpec((_GW, _EMB), index_map=lambda i: (i, 0))],
            core_axis_name=("core", "subcore"),
            dimension_semantics=(pltpu.PARALLEL,),
        )(idx_hbm, out_hbm)

    emb = gather_rows(table, idx)

    out = pl.pallas_call(
        _matmul_relu_block,
        grid=(n // _BM,),
        in_specs=[
            pl.BlockSpec((_BM, _EMB), lambda i: (i, 0)),
            pl.BlockSpec((_EMB, _LIN), lambda i: (0, 0)),
            pl.BlockSpec((1, _LIN), lambda i: (0, 0)),
        ],
        out_specs=pl.BlockSpec((_BM, _LIN), lambda i: (i, 0)),
        out_shape=jax.ShapeDtypeStruct((n, _LIN), jnp.float32),
    )(emb, W1, b1.reshape(1, _LIN))

    return out.reshape(B, L, _LIN)


# speedup vs baseline: 1.8471x; 1.2420x over previous
import jax
import jax.numpy as jnp
from jax.experimental import pallas as pl
from jax.experimental.pallas import tpu as pltpu
from jax.experimental.pallas import tpu_sc as plsc

_EMB = 128
_LIN = 256
_GW = 128
_BB = 16


def _matmul_relu_block(x_ref, w_ref, b_ref, o_ref):
    acc = jnp.dot(x_ref[...], w_ref[...], preferred_element_type=jnp.float32)
    res = jnp.maximum(acc + b_ref[...], 0.0)
    o_ref[...] = res.reshape(o_ref.shape)


def kernel(inputs, table, W1, b1):
    B, L = inputs.shape
    n = B * L
    idx = inputs.reshape(1, n).astype(jnp.int32)

    mesh = plsc.VectorSubcoreMesh(core_axis_name="core", subcore_axis_name="subcore")

    @pl.kernel(out_type=jax.ShapeDtypeStruct((n, _EMB), table.dtype), mesh=mesh)
    def gather_rows(table_hbm, idx_hbm, out_hbm):
        def body(idx_vmem, out_vmem):
            pltpu.sync_copy(table_hbm.at[idx_vmem.at[0]], out_vmem)

        pltpu.emit_pipeline(
            body,
            grid=(n // _GW,),
            in_specs=[pl.BlockSpec((1, _GW), index_map=lambda i: (0, i))],
            out_specs=[pl.BlockSpec((_GW, _EMB), index_map=lambda i: (i, 0))],
            core_axis_name=("core", "subcore"),
            dimension_semantics=(pltpu.PARALLEL,),
        )(idx_hbm, out_hbm)

    emb = gather_rows(table, idx)

    out = pl.pallas_call(
        _matmul_relu_block,
        grid=(B // _BB,),
        in_specs=[
            pl.BlockSpec((_BB * L, _EMB), lambda i: (i, 0)),
            pl.BlockSpec((_EMB, _LIN), lambda i: (0, 0)),
            pl.BlockSpec((1, _LIN), lambda i: (0, 0)),
        ],
        out_specs=pl.BlockSpec((_BB, L, _LIN), lambda i: (i, 0, 0)),
        out_shape=jax.ShapeDtypeStruct((B, L, _LIN), jnp.float32),
    )(emb, W1, b1.reshape(1, _LIN))

    return out


# speedup vs baseline: 3.6099x; 1.9544x over previous
import jax
import jax.numpy as jnp
from jax.experimental import pallas as pl
from jax.experimental.pallas import tpu as pltpu
from jax.experimental.pallas import tpu_sc as plsc

_EMB = 128
_LIN = 256
_GW = 128
_BM = 2048


def _matmul_relu_block(x_ref, w_ref, b_ref, o_ref):
    acc = jnp.dot(x_ref[...], w_ref[...], preferred_element_type=jnp.float32)
    o_ref[...] = jnp.maximum(acc + b_ref[...], 0.0)


def kernel(inputs, table, W1, b1):
    B, L = inputs.shape
    n = B * L
    idx = inputs.T.reshape(1, n).astype(jnp.int32)

    mesh = plsc.VectorSubcoreMesh(core_axis_name="core", subcore_axis_name="subcore")

    @pl.kernel(out_type=jax.ShapeDtypeStruct((n, _EMB), table.dtype), mesh=mesh)
    def gather_rows(table_hbm, idx_hbm, out_hbm):
        def body(idx_vmem, out_vmem):
            pltpu.sync_copy(table_hbm.at[idx_vmem.at[0]], out_vmem)

        pltpu.emit_pipeline(
            body,
            grid=(n // _GW,),
            in_specs=[pl.BlockSpec((1, _GW), index_map=lambda i: (0, i))],
            out_specs=[pl.BlockSpec((_GW, _EMB), index_map=lambda i: (i, 0))],
            core_axis_name=("core", "subcore"),
            dimension_semantics=(pltpu.PARALLEL,),
        )(idx_hbm, out_hbm)

    emb = gather_rows(table, idx)

    out = pl.pallas_call(
        _matmul_relu_block,
        grid=(n // _BM,),
        in_specs=[
            pl.BlockSpec((_BM, _EMB), lambda i: (i, 0)),
            pl.BlockSpec((_EMB, _LIN), lambda i: (0, 0)),
            pl.BlockSpec((1, _LIN), lambda i: (0, 0)),
        ],
        out_specs=pl.BlockSpec((_BM, _LIN), lambda i: (i, 0)),
        out_shape=jax.ShapeDtypeStruct((n, _LIN), jnp.float32),
    )(emb, W1, b1.reshape(1, _LIN))

    return out.reshape(L, B, _LIN).transpose(1, 0, 2)


# speedup vs baseline: 4.1184x; 1.1408x over previous
import jax
import jax.numpy as jnp
from jax.experimental import pallas as pl
from jax.experimental.pallas import tpu as pltpu
from jax.experimental.pallas import tpu_sc as plsc

_EMB = 128
_LIN = 256
_GW = 128
_BM = 4096


def _matmul_relu_block(x_ref, w_ref, b_ref, o_ref):
    acc = jnp.dot(x_ref[...], w_ref[...], preferred_element_type=jnp.float32)
    o_ref[...] = jnp.maximum(acc + b_ref[...], 0.0)


def kernel(inputs, table, W1, b1):
    B, L = inputs.shape
    n = B * L
    idx = inputs.T.reshape(1, n).astype(jnp.int32)

    mesh = plsc.VectorSubcoreMesh(core_axis_name="core", subcore_axis_name="subcore")

    @pl.kernel(out_type=jax.ShapeDtypeStruct((n, _EMB), table.dtype), mesh=mesh)
    def gather_rows(table_hbm, idx_hbm, out_hbm):
        def body(idx_vmem, out_vmem):
            pltpu.sync_copy(table_hbm.at[idx_vmem.at[0]], out_vmem)

        pltpu.emit_pipeline(
            body,
            grid=(n // _GW,),
            in_specs=[pl.BlockSpec((1, _GW), index_map=lambda i: (0, i))],
            out_specs=[pl.BlockSpec((_GW, _EMB), index_map=lambda i: (i, 0))],
            core_axis_name=("core", "subcore"),
            dimension_semantics=(pltpu.PARALLEL,),
        )(idx_hbm, out_hbm)

    emb = gather_rows(table, idx)

    out = pl.pallas_call(
        _matmul_relu_block,
        grid=(n // _BM,),
        in_specs=[
            pl.BlockSpec((_BM, _EMB), lambda i: (i, 0)),
            pl.BlockSpec((_EMB, _LIN), lambda i: (0, 0)),
            pl.BlockSpec((1, _LIN), lambda i: (0, 0)),
        ],
        out_specs=pl.BlockSpec((_BM, _LIN), lambda i: (i, 0)),
        out_shape=jax.ShapeDtypeStruct((n, _LIN), jnp.float32),
    )(emb, W1, b1.reshape(1, _LIN))

    return out.reshape(L, B, _LIN).transpose(1, 0, 2)


# speedup vs baseline: 4.2328x; 1.0278x over previous
import jax
import jax.numpy as jnp
from jax.experimental import pallas as pl
from jax.experimental.pallas import tpu as pltpu
from jax.experimental.pallas import tpu_sc as plsc

_EMB = 128
_LIN = 256
_GW = 128
_BM = 8192


def _matmul_relu_block(x_ref, w_ref, b_ref, o_ref):
    acc = jnp.dot(x_ref[...], w_ref[...], preferred_element_type=jnp.float32)
    o_ref[...] = jnp.maximum(acc + b_ref[...], 0.0)


def kernel(inputs, table, W1, b1):
    B, L = inputs.shape
    n = B * L
    idx = inputs.T.reshape(1, n).astype(jnp.int32)

    mesh = plsc.VectorSubcoreMesh(core_axis_name="core", subcore_axis_name="subcore")

    @pl.kernel(out_type=jax.ShapeDtypeStruct((n, _EMB), table.dtype), mesh=mesh)
    def gather_rows(table_hbm, idx_hbm, out_hbm):
        def body(idx_vmem, out_vmem):
            pltpu.sync_copy(table_hbm.at[idx_vmem.at[0]], out_vmem)

        pltpu.emit_pipeline(
            body,
            grid=(n // _GW,),
            in_specs=[pl.BlockSpec((1, _GW), index_map=lambda i: (0, i))],
            out_specs=[pl.BlockSpec((_GW, _EMB), index_map=lambda i: (i, 0))],
            core_axis_name=("core", "subcore"),
            dimension_semantics=(pltpu.PARALLEL,),
        )(idx_hbm, out_hbm)

    emb = gather_rows(table, idx)

    out = pl.pallas_call(
        _matmul_relu_block,
        grid=(n // _BM,),
        in_specs=[
            pl.BlockSpec((_BM, _EMB), lambda i: (i, 0)),
            pl.BlockSpec((_EMB, _LIN), lambda i: (0, 0)),
            pl.BlockSpec((1, _LIN), lambda i: (0, 0)),
        ],
        out_specs=pl.BlockSpec((_BM, _LIN), lambda i: (i, 0)),
        out_shape=jax.ShapeDtypeStruct((n, _LIN), jnp.float32),
    )(emb, W1, b1.reshape(1, _LIN))

    return out.reshape(L, B, _LIN).transpose(1, 0, 2)


# speedup vs baseline: 4.3691x; 1.0322x over previous
import jax
import jax.numpy as jnp
from jax.experimental import pallas as pl
from jax.experimental.pallas import tpu as pltpu
from jax.experimental.pallas import tpu_sc as plsc

_EMB = 128
_LIN = 256
_GW = 128
_BM = 16384


def _matmul_relu_block(x_ref, w_ref, b_ref, o_ref):
    acc = jnp.dot(x_ref[...], w_ref[...], preferred_element_type=jnp.float32)
    o_ref[...] = jnp.maximum(acc + b_ref[...], 0.0)


def kernel(inputs, table, W1, b1):
    B, L = inputs.shape
    n = B * L
    idx = inputs.T.reshape(1, n).astype(jnp.int32)

    mesh = plsc.VectorSubcoreMesh(core_axis_name="core", subcore_axis_name="subcore")

    @pl.kernel(out_type=jax.ShapeDtypeStruct((n, _EMB), table.dtype), mesh=mesh)
    def gather_rows(table_hbm, idx_hbm, out_hbm):
        def body(idx_vmem, out_vmem):
            pltpu.sync_copy(table_hbm.at[idx_vmem.at[0]], out_vmem)

        pltpu.emit_pipeline(
            body,
            grid=(n // _GW,),
            in_specs=[pl.BlockSpec((1, _GW), index_map=lambda i: (0, i))],
            out_specs=[pl.BlockSpec((_GW, _EMB), index_map=lambda i: (i, 0))],
            core_axis_name=("core", "subcore"),
            dimension_semantics=(pltpu.PARALLEL,),
        )(idx_hbm, out_hbm)

    emb = gather_rows(table, idx)

    out = pl.pallas_call(
        _matmul_relu_block,
        grid=(n // _BM,),
        in_specs=[
            pl.BlockSpec((_BM, _EMB), lambda i: (i, 0)),
            pl.BlockSpec((_EMB, _LIN), lambda i: (0, 0)),
            pl.BlockSpec((1, _LIN), lambda i: (0, 0)),
        ],
        out_specs=pl.BlockSpec((_BM, _LIN), lambda i: (i, 0)),
        out_shape=jax.ShapeDtypeStruct((n, _LIN), jnp.float32),
    )(emb, W1, b1.reshape(1, _LIN))

    return out.reshape(L, B, _LIN).transpose(1, 0, 2)
